# baseline (speedup 1.0000x reference)
import functools

import jax
import jax.numpy as jnp
from jax import lax
from jax.experimental import pallas as pl
from jax.experimental.pallas import tpu as pltpu
from jax.experimental.pallas import tpu_sc as plsc

NATTRS = 115
NOBJS = 245
NOBJ_PAD = 256
ATTR_BLK = 8
NATTR_PAD = 120
N_NODES = NATTRS + NOBJS
N_EDGES = 60000
FLAT = N_NODES * N_NODES

P_TILES = 12
EPT = 5120
NE_PAD = P_TILES * EPT
CH = 256
NCH = EPT // CH
GRP = CH // 16

_EPS = 1e-5


def _vgather16(x, idx):
    dn = lax.GatherDimensionNumbers(
        offset_dims=(), collapsed_slice_dims=(0,), start_index_map=(0,))
    return lax.gather(x, idx[:, None], dn, slice_sizes=(1,),
                      mode=lax.GatherScatterMode.PROMISE_IN_BOUNDS)


def _adj_body(edges_hbm, zeros_hbm, out_hbm, ev0, ev1, acc, semz, sem0, sem1):
    c = lax.axis_index("c")
    s = lax.axis_index("s")
    wid = s * 2 + c

    @pl.when(wid < P_TILES)
    def _():
        zdma = pltpu.make_async_copy(zeros_hbm, acc, semz)
        zdma.start()
        lane = lax.iota(jnp.int32, 16)
        evs = [ev0, ev1]
        sems = [sem0, sem1]

        def start_chunk(ch):
            ebase = wid * EPT + ch * CH
            d = pltpu.make_async_copy(
                edges_hbm.at[:, pl.ds(ebase, CH)], evs[ch % 2], sems[ch % 2])
            d.start()
            return d

        dma = start_chunk(0)
        zdma.wait()
        for ch in range(NCH):
            nxt_dma = start_chunk(ch + 1) if ch + 1 < NCH else None
            dma.wait()
            ev = evs[ch % 2]
            ebase = wid * EPT + ch * CH

            def grp(g, carry):
                off = g * 16
                src = ev[0, pl.ds(off, 16)]
                dst = ev[1, pl.ds(off, 16)]
                flat = dst * N_NODES + src
                val = jnp.where(ebase + off + lane < N_EDGES, 1.0, 0.0)
                skey, sval = plsc.sort_key_val(flat, val)
                csum = plsc.cumsum(sval)
                nxt = _vgather16(skey, jnp.minimum(lane + 1, 15))
                is_last = (skey != nxt) | (lane == 15)
                bounded = jnp.where(is_last, csum, 0.0)
                bmax = plsc.cummax(bounded)
                prev = jnp.where(lane == 0, 0.0,
                                 _vgather16(bmax, jnp.maximum(lane - 1, 0)))
                run = csum - prev
                plsc.addupdate_scatter(acc, [skey], run, mask=is_last)
                return carry

            lax.fori_loop(0, GRP, grp, 0)
            dma = nxt_dma
        pltpu.sync_copy(acc, out_hbm.at[wid])


@functools.cache
def _adj_kernel_fn():
    return pl.kernel(
        _adj_body,
        out_type=jax.ShapeDtypeStruct((P_TILES, FLAT), jnp.float32),
        mesh=plsc.VectorSubcoreMesh(core_axis_name="c", subcore_axis_name="s"),
        scratch_types=[
            pltpu.VMEM((2, CH), jnp.int32),
            pltpu.VMEM((2, CH), jnp.int32),
            pltpu.VMEM((FLAT,), jnp.float32),
            pltpu.SemaphoreType.DMA,
            pltpu.SemaphoreType.DMA,
            pltpu.SemaphoreType.DMA,
        ],
        compiler_params=pltpu.CompilerParams(needs_layout_passes=False),
    )


def _ln(x, g, b):
    mu = jnp.mean(x, axis=-1, keepdims=True)
    xc = x - mu
    var = jnp.mean(xc * xc, axis=-1, keepdims=True)
    return xc * lax.rsqrt(var + _EPS) * g + b


def _fused_body(pref, nodes_ref, wg1l_ref, bg1_ref, wg1r_ref,
                wg2l_ref, bg2_ref, wg2r_ref,
                wp1a_ref, wp1b_ref,
                ximg_ref, wi1_ref, bi1_ref, gi1_ref, bei1_ref,
                wi2_ref, bi2_ref, gi2_ref, bei2_ref,
                wi3_ref, bi3_ref, gio_ref, beio_ref,
                bp1_ref, gp1_ref, bep1_ref,
                wp2_ref, bp2_ref, gpo_ref, bepo_ref,
                out_ref, ctop_s, b1_s, femb_s):
    f32 = jnp.float32
    bf16 = jnp.bfloat16
    cdims = (((1,), (1,)), ((), ()))
    i = pl.program_id(0)

    @pl.when(i == 0)
    def _prep():
        A = jnp.sum(pref[...], axis=0)
        deg = jnp.sum(A, axis=1, keepdims=True)
        An = A / jnp.maximum(deg, 1.0)
        nodes = nodes_ref[...]
        P1 = jnp.dot(An, nodes, preferred_element_type=f32)
        h = jnp.dot(P1, wg1l_ref[...], preferred_element_type=f32)
        h = h + bg1_ref[...]
        h = h + jnp.dot(nodes, wg1r_ref[...], preferred_element_type=f32)
        h = jnp.maximum(h, 0.0)
        Q = jnp.dot(An, h, preferred_element_type=f32)
        emb = jnp.dot(Q, wg2l_ref[...], preferred_element_type=f32)
        emb = emb + bg2_ref[...]
        emb = emb + jnp.dot(h, wg2r_ref[...], preferred_element_type=f32)
        ctop = jnp.dot(emb[:NATTRS], wp1a_ref[...], preferred_element_type=f32)
        cbot = jnp.dot(emb[NATTRS:], wp1b_ref[...], preferred_element_type=f32)
        ctop_s[...] = jnp.concatenate(
            [ctop, jnp.zeros((NATTR_PAD - NATTRS, 1000), f32)], axis=0)
        b1_s[...] = jnp.concatenate(
            [cbot + bp1_ref[...], jnp.zeros((NOBJ_PAD - NOBJS, 1000), f32)],
            axis=0)
        f = jnp.dot(ximg_ref[...], wi1_ref[...], preferred_element_type=f32)
        f = jnp.maximum(_ln(f + bi1_ref[...], gi1_ref[...], bei1_ref[...]),
                        0.0)
        f = jnp.dot(f, wi2_ref[...], preferred_element_type=f32)
        f = jnp.maximum(_ln(f + bi2_ref[...], gi2_ref[...], bei2_ref[...]),
                        0.0)
        f = jnp.dot(f, wi3_ref[...], preferred_element_type=f32)
        femb_s[...] = _ln(f + bi3_ref[...], gio_ref[...], beio_ref[...])

    @pl.when(i > 0)
    def _pairs():
        j = i - 1
        b1 = b1_s[...]
        a_blk = ctop_s[pl.ds(j * ATTR_BLK, ATTR_BLK), :]
        z = jnp.concatenate(
            [b1 + a_blk[k:k + 1, :] for k in range(ATTR_BLK)], axis=0)
        q = jnp.maximum(_ln(z, gp1_ref[...], bep1_ref[...]), 0.0).astype(bf16)
        y = jnp.dot(q, wp2_ref[...].astype(bf16),
                    preferred_element_type=f32) + bp2_ref[...]
        p = _ln(y, gpo_ref[...], bepo_ref[...])
        out_ref[...] = lax.dot_general(
            femb_s[...].astype(bf16), p.astype(bf16), cdims,
            preferred_element_type=f32)


def kernel(x_img, nodes, edge_index, W_g1l, b_g1, W_g1r, W_g2l, b_g2, W_g2r,
           Wi1, bi1, gi1, bei1, Wi2, bi2, gi2, bei2, Wi3, bi3, gio, beio,
           Wp1, bp1, gp1, bep1, Wp2, bp2, gpo, bepo):
    f32 = jnp.float32
    r1 = lambda v: v.reshape(1, -1)

    epad = jnp.pad(edge_index, ((0, 0), (0, NE_PAD - N_EDGES)))
    partials = _adj_kernel_fn()(epad, jnp.zeros((FLAT,), f32))
    partials = partials.reshape(P_TILES, N_NODES, N_NODES)

    cspec = lambda shape: pl.BlockSpec(shape, lambda i: tuple(0 for _ in shape))
    grid = (1 + NATTR_PAD // ATTR_BLK,)
    out = pl.pallas_call(
        _fused_body,
        grid=grid,
        in_specs=[
            cspec((P_TILES, N_NODES, N_NODES)),
            cspec((N_NODES, 512)),
            cspec((512, 2048)), cspec((1, 2048)), cspec((512, 2048)),
            cspec((2048, 800)), cspec((1, 800)), cspec((2048, 800)),
            cspec((800, 1000)), cspec((800, 1000)),
            cspec((128, 512)),
            cspec((512, 800)), cspec((1, 800)), cspec((1, 800)),
            cspec((1, 800)),
            cspec((800, 1000)), cspec((1, 1000)), cspec((1, 1000)),
            cspec((1, 1000)),
            cspec((1000, 800)), cspec((1, 800)), cspec((1, 800)),
            cspec((1, 800)),
            cspec((1, 1000)), cspec((1, 1000)), cspec((1, 1000)),
            cspec((1000, 800)), cspec((1, 800)), cspec((1, 800)),
            cspec((1, 800)),
        ],
        out_specs=pl.BlockSpec((128, ATTR_BLK * NOBJ_PAD),
                               lambda i: (0, jnp.maximum(i - 1, 0))),
        out_shape=jax.ShapeDtypeStruct((128, NATTR_PAD * NOBJ_PAD), f32),
        scratch_shapes=[
            pltpu.VMEM((NATTR_PAD, 1000), f32),
            pltpu.VMEM((NOBJ_PAD, 1000), f32),
            pltpu.VMEM((128, 800), f32),
        ],
        compiler_params=pltpu.CompilerParams(
            vmem_limit_bytes=100 * 1024 * 1024),
    )(partials, nodes,
      W_g1l, r1(b_g1), W_g1r, W_g2l, r1(b_g2), W_g2r,
      Wp1[:800], Wp1[800:],
      x_img, Wi1, r1(bi1), r1(gi1), r1(bei1),
      Wi2, r1(bi2), r1(gi2), r1(bei2),
      Wi3, r1(bi3), r1(gio), r1(beio),
      r1(bp1), r1(gp1), r1(bep1),
      Wp2, r1(bp2), r1(gpo), r1(bepo))
    out = out.reshape(128, NATTR_PAD, NOBJ_PAD)[:, :NATTRS, :NOBJS]
    return out.reshape(128, NATTRS * NOBJS)


# speedup vs baseline: 1.0439x; 1.0439x over previous
import functools

import jax
import jax.numpy as jnp
from jax import lax
from jax.experimental import pallas as pl
from jax.experimental.pallas import tpu as pltpu
from jax.experimental.pallas import tpu_sc as plsc

NATTRS = 115
NOBJS = 245
NOBJ_PAD = 256
ATTR_BLK = 5
N_ABLK = NATTRS // ATTR_BLK
N_NODES = NATTRS + NOBJS
N_EDGES = 60000
FLAT = N_NODES * N_NODES

P_TILES = 12
EPT = 5120
NE_PAD = P_TILES * EPT
CH = 256
NCH = EPT // CH
GRP = CH // 16

_EPS = 1e-5


def _vgather16(x, idx):
    dn = lax.GatherDimensionNumbers(
        offset_dims=(), collapsed_slice_dims=(0,), start_index_map=(0,))
    return lax.gather(x, idx[:, None], dn, slice_sizes=(1,),
                      mode=lax.GatherScatterMode.PROMISE_IN_BOUNDS)


def _adj_body(edges_hbm, zeros_hbm, out_hbm, ev0, ev1, acc, semz, sem0, sem1):
    c = lax.axis_index("c")
    s = lax.axis_index("s")
    wid = s * 2 + c

    @pl.when(wid < P_TILES)
    def _():
        zdma = pltpu.make_async_copy(zeros_hbm, acc, semz)
        zdma.start()
        lane = lax.iota(jnp.int32, 16)
        evs = [ev0, ev1]
        sems = [sem0, sem1]

        def start_chunk(ch):
            ebase = wid * EPT + ch * CH
            d = pltpu.make_async_copy(
                edges_hbm.at[:, pl.ds(ebase, CH)], evs[ch % 2], sems[ch % 2])
            d.start()
            return d

        dma = start_chunk(0)
        zdma.wait()
        for ch in range(NCH):
            nxt_dma = start_chunk(ch + 1) if ch + 1 < NCH else None
            dma.wait()
            ev = evs[ch % 2]
            ebase = wid * EPT + ch * CH

            def grp(g, carry):
                off = g * 16
                src = ev[0, pl.ds(off, 16)]
                dst = ev[1, pl.ds(off, 16)]
                flat = dst * N_NODES + src
                val = jnp.where(ebase + off + lane < N_EDGES, 1.0, 0.0)
                skey, sval = plsc.sort_key_val(flat, val)
                csum = plsc.cumsum(sval)
                nxt = _vgather16(skey, jnp.minimum(lane + 1, 15))
                is_last = (skey != nxt) | (lane == 15)
                bounded = jnp.where(is_last, csum, 0.0)
                bmax = plsc.cummax(bounded)
                prev = jnp.where(lane == 0, 0.0,
                                 _vgather16(bmax, jnp.maximum(lane - 1, 0)))
                run = csum - prev
                plsc.addupdate_scatter(acc, [skey], run, mask=is_last)
                return carry

            lax.fori_loop(0, GRP, grp, 0)
            dma = nxt_dma
        pltpu.sync_copy(acc, out_hbm.at[wid])


@functools.cache
def _adj_kernel_fn():
    return pl.kernel(
        _adj_body,
        out_type=jax.ShapeDtypeStruct((P_TILES, FLAT), jnp.float32),
        mesh=plsc.VectorSubcoreMesh(core_axis_name="c", subcore_axis_name="s"),
        scratch_types=[
            pltpu.VMEM((2, CH), jnp.int32),
            pltpu.VMEM((2, CH), jnp.int32),
            pltpu.VMEM((FLAT,), jnp.float32),
            pltpu.SemaphoreType.DMA,
            pltpu.SemaphoreType.DMA,
            pltpu.SemaphoreType.DMA,
        ],
        compiler_params=pltpu.CompilerParams(needs_layout_passes=False),
    )


def _ln(x, g, b):
    mu = jnp.mean(x, axis=-1, keepdims=True)
    xc = x - mu
    var = jnp.mean(xc * xc, axis=-1, keepdims=True)
    return xc * lax.rsqrt(var + _EPS) * g + b


def _fused_body(pref, nodes_ref, wg1l_ref, bg1_ref, wg1r_ref,
                wg2l_ref, bg2_ref, wg2r_ref,
                wp1a_ref, wp1b_ref,
                ximg_ref, wi1_ref, bi1_ref, gi1_ref, bei1_ref,
                wi2_ref, bi2_ref, gi2_ref, bei2_ref,
                wi3_ref, bi3_ref, gio_ref, beio_ref,
                bp1_ref, gp1_ref, bep1_ref,
                wp2_ref, bp2_ref, gpo_ref, bepo_ref,
                out_ref, ctop_s, b1_s, femb_s):
    f32 = jnp.float32
    bf16 = jnp.bfloat16
    cdims = (((1,), (1,)), ((), ()))
    i = pl.program_id(0)

    @pl.when(i == 0)
    def _prep():
        A = jnp.sum(pref[...], axis=0)
        deg = jnp.sum(A, axis=1, keepdims=True)
        An = A / jnp.maximum(deg, 1.0)
        nodes = nodes_ref[...]
        P1 = jnp.dot(An, nodes, preferred_element_type=f32)
        h = jnp.dot(P1, wg1l_ref[...], preferred_element_type=f32)
        h = h + bg1_ref[...]
        h = h + jnp.dot(nodes, wg1r_ref[...], preferred_element_type=f32)
        h = jnp.maximum(h, 0.0)
        Q = jnp.dot(An, h, preferred_element_type=f32)
        emb = jnp.dot(Q, wg2l_ref[...], preferred_element_type=f32)
        emb = emb + bg2_ref[...]
        emb = emb + jnp.dot(h, wg2r_ref[...], preferred_element_type=f32)
        ctop = jnp.dot(emb[:NATTRS], wp1a_ref[...], preferred_element_type=f32)
        cbot = jnp.dot(emb[NATTRS:], wp1b_ref[...], preferred_element_type=f32)
        for blk in range(N_ABLK):
            ctop_s[blk, pl.ds(0, ATTR_BLK), :] = (
                ctop[blk * ATTR_BLK:(blk + 1) * ATTR_BLK])
        b1_s[...] = jnp.concatenate(
            [cbot + bp1_ref[...], jnp.zeros((NOBJ_PAD - NOBJS, 1000), f32)],
            axis=0)
        f = jnp.dot(ximg_ref[...], wi1_ref[...], preferred_element_type=f32)
        f = jnp.maximum(_ln(f + bi1_ref[...], gi1_ref[...], bei1_ref[...]),
                        0.0)
        f = jnp.dot(f, wi2_ref[...], preferred_element_type=f32)
        f = jnp.maximum(_ln(f + bi2_ref[...], gi2_ref[...], bei2_ref[...]),
                        0.0)
        f = jnp.dot(f, wi3_ref[...], preferred_element_type=f32)
        femb_s[...] = _ln(f + bi3_ref[...], gio_ref[...], beio_ref[...])

    @pl.when(i > 0)
    def _pairs():
        j = i - 1
        b1 = b1_s[...]
        a_blk = ctop_s[j, pl.ds(0, ATTR_BLK), :]
        z = jnp.concatenate(
            [b1 + a_blk[k:k + 1, :] for k in range(ATTR_BLK)], axis=0)
        q = jnp.maximum(_ln(z, gp1_ref[...], bep1_ref[...]), 0.0).astype(bf16)
        y = jnp.dot(q, wp2_ref[...].astype(bf16),
                    preferred_element_type=f32) + bp2_ref[...]
        p = _ln(y, gpo_ref[...], bepo_ref[...])
        out_ref[...] = lax.dot_general(
            femb_s[...].astype(bf16), p.astype(bf16), cdims,
            preferred_element_type=f32)


def kernel(x_img, nodes, edge_index, W_g1l, b_g1, W_g1r, W_g2l, b_g2, W_g2r,
           Wi1, bi1, gi1, bei1, Wi2, bi2, gi2, bei2, Wi3, bi3, gio, beio,
           Wp1, bp1, gp1, bep1, Wp2, bp2, gpo, bepo):
    f32 = jnp.float32
    r1 = lambda v: v.reshape(1, -1)

    epad = jnp.pad(edge_index, ((0, 0), (0, NE_PAD - N_EDGES)))
    partials = _adj_kernel_fn()(epad, jnp.zeros((FLAT,), f32))
    partials = partials.reshape(P_TILES, N_NODES, N_NODES)

    cspec = lambda shape: pl.BlockSpec(shape, lambda i: tuple(0 for _ in shape))
    grid = (1 + N_ABLK,)
    out = pl.pallas_call(
        _fused_body,
        grid=grid,
        in_specs=[
            cspec((P_TILES, N_NODES, N_NODES)),
            cspec((N_NODES, 512)),
            cspec((512, 2048)), cspec((1, 2048)), cspec((512, 2048)),
            cspec((2048, 800)), cspec((1, 800)), cspec((2048, 800)),
            cspec((800, 1000)), cspec((800, 1000)),
            cspec((128, 512)),
            cspec((512, 800)), cspec((1, 800)), cspec((1, 800)),
            cspec((1, 800)),
            cspec((800, 1000)), cspec((1, 1000)), cspec((1, 1000)),
            cspec((1, 1000)),
            cspec((1000, 800)), cspec((1, 800)), cspec((1, 800)),
            cspec((1, 800)),
            cspec((1, 1000)), cspec((1, 1000)), cspec((1, 1000)),
            cspec((1000, 800)), cspec((1, 800)), cspec((1, 800)),
            cspec((1, 800)),
        ],
        out_specs=pl.BlockSpec((128, ATTR_BLK * NOBJ_PAD),
                               lambda i: (0, jnp.maximum(i - 1, 0))),
        out_shape=jax.ShapeDtypeStruct((128, NATTRS * NOBJ_PAD), f32),
        scratch_shapes=[
            pltpu.VMEM((N_ABLK, 8, 1000), f32),
            pltpu.VMEM((NOBJ_PAD, 1000), f32),
            pltpu.VMEM((128, 800), f32),
        ],
        compiler_params=pltpu.CompilerParams(
            vmem_limit_bytes=100 * 1024 * 1024),
    )(partials, nodes,
      W_g1l, r1(b_g1), W_g1r, W_g2l, r1(b_g2), W_g2r,
      Wp1[:800], Wp1[800:],
      x_img, Wi1, r1(bi1), r1(gi1), r1(bei1),
      Wi2, r1(bi2), r1(gi2), r1(bei2),
      Wi3, r1(bi3), r1(gio), r1(beio),
      r1(bp1), r1(gp1), r1(bep1),
      Wp2, r1(bp2), r1(gpo), r1(bepo))
    out = out.reshape(128, NATTRS, NOBJ_PAD)[:, :, :NOBJS]
    return out.reshape(128, NATTRS * NOBJS)
